# Initial kernel scaffold; baseline (speedup 1.0000x reference)
import jax
import jax.numpy as jnp
from jax.experimental import pallas as pl


def kernel(x, weight, bias):
    raise NotImplementedError("write your pallas kernel here")


# speedup vs baseline: 5.3624x; 5.3624x over previous
import functools

import jax
import jax.numpy as jnp
from jax.experimental import pallas as pl
from jax.experimental.pallas import tpu as pltpu


def _gtu_body(x_ref, w_ref, b_ref, o_ref, *, C, K, TN, T, T_out):
    L = TN * T
    xb = x_ref[0].astype(jnp.bfloat16)
    xb = xb.reshape(C, L)
    parts = [xb] + [jnp.roll(xb, -k, axis=1) for k in range(1, K)]
    xp = jnp.concatenate(parts, axis=0)
    r = jnp.dot(w_ref[...], xp,
                preferred_element_type=jnp.float32)
    r = r + b_ref[...]
    g = jnp.tanh(r[:C]) * jax.nn.sigmoid(r[C:])
    o_ref[0] = g.reshape(C, TN, T)[:, :, :T_out]


def kernel(x, weight, bias):
    B, C, N, T = x.shape
    K = weight.shape[-1]
    C2 = 2 * C
    T_out = T - K + 1

    TN = 128

    w2 = jnp.transpose(weight[:, :, 0, :], (0, 2, 1)).reshape(C2, K * C)
    w2 = w2.astype(jnp.bfloat16)
    b2 = bias.reshape(C2, 1)

    body = functools.partial(_gtu_body, C=C, K=K, TN=TN, T=T, T_out=T_out)
    return pl.pallas_call(
        body,
        out_shape=jax.ShapeDtypeStruct((B, C, N, T_out), jnp.float32),
        grid=(B, N // TN),
        in_specs=[
            pl.BlockSpec((1, C, TN, T), lambda b, j: (b, 0, j, 0)),
            pl.BlockSpec((C2, K * C), lambda b, j: (0, 0)),
            pl.BlockSpec((C2, 1), lambda b, j: (0, 0)),
        ],
        out_specs=pl.BlockSpec((1, C, TN, T_out), lambda b, j: (b, 0, j, 0)),
        compiler_params=pltpu.CompilerParams(
            dimension_semantics=("parallel", "parallel")),
    )(x, w2, b2)


# speedup vs baseline: 5.5515x; 1.0353x over previous
import functools

import jax
import jax.numpy as jnp
from jax.experimental import pallas as pl
from jax.experimental.pallas import tpu as pltpu


def _gtu_body(x_ref, w_ref, b_ref, o_ref, *, C, K, TN, T, T_out):
    L = TN * T
    xb = x_ref[0].astype(jnp.bfloat16)
    xb = xb.reshape(C, L)
    parts = [xb] + [jnp.roll(xb, -k, axis=1) for k in range(1, K)]
    xp = jnp.concatenate(parts, axis=0)
    r = jnp.dot(w_ref[...], xp,
                preferred_element_type=jnp.float32)
    r = r + b_ref[...]
    g = jnp.tanh(r[:C]) * (0.5 + 0.5 * jnp.tanh(r[C:]))
    o_ref[0] = g.reshape(C, TN, T)[:, :, :T_out]


def kernel(x, weight, bias):
    B, C, N, T = x.shape
    K = weight.shape[-1]
    C2 = 2 * C
    T_out = T - K + 1

    TN = 128

    w2 = jnp.transpose(weight[:, :, 0, :], (0, 2, 1)).reshape(C2, K * C)
    scale = jnp.concatenate([jnp.ones((C, 1)), jnp.full((C, 1), 0.5)], axis=0)
    w2 = (w2 * scale).astype(jnp.bfloat16)
    b2 = bias.reshape(C2, 1) * scale

    body = functools.partial(_gtu_body, C=C, K=K, TN=TN, T=T, T_out=T_out)
    return pl.pallas_call(
        body,
        out_shape=jax.ShapeDtypeStruct((B, C, N, T_out), jnp.float32),
        grid=(B, N // TN),
        in_specs=[
            pl.BlockSpec((1, C, TN, T), lambda b, j: (b, 0, j, 0)),
            pl.BlockSpec((C2, K * C), lambda b, j: (0, 0)),
            pl.BlockSpec((C2, 1), lambda b, j: (0, 0)),
        ],
        out_specs=pl.BlockSpec((1, C, TN, T_out), lambda b, j: (b, 0, j, 0)),
        compiler_params=pltpu.CompilerParams(
            dimension_semantics=("parallel", "parallel")),
    )(x, w2, b2)
